# baseline (speedup 1.0000x reference)
import functools

import jax
import jax.numpy as jnp
from jax import lax
from jax.experimental import pallas as pl
from jax.experimental.pallas import tpu as pltpu
from jax.experimental.pallas import tpu_sc as plsc

N = 10000
E = 320000
D = 128
NTILES = 16
NWORKERS = 32
CHUNK = 40
EPT = E // NWORKERS
NCHUNK = EPT // CHUNK
ROWS_PER_TILE = N // NTILES
RING = 5
LOOKAHEAD = 3


def _sc_segment_sum(feature, edge_index):
    mesh = plsc.VectorSubcoreMesh(core_axis_name="c", subcore_axis_name="s")

    @functools.partial(
        pl.kernel,
        mesh=mesh,
        compiler_params=pltpu.CompilerParams(use_tc_tiling_on_sc=False),
        out_type=jax.ShapeDtypeStruct((2, N, D), jnp.float32),
        scratch_types=[
            pltpu.VMEM_SHARED((N, D), jnp.float32),
            pltpu.VMEM((EPT,), jnp.int32),
            pltpu.VMEM((EPT,), jnp.int32),
            [pltpu.VMEM((CHUNK, D), jnp.float32)] * RING,
            [pltpu.SemaphoreType.DMA] * RING,
            [pltpu.SemaphoreType.DMA] * RING,
        ],
    )
    def k(feat_hbm, edge_hbm, out_hbm, acc_sh, src_v, dst_v, bufs, sg, ss):
        cid = lax.axis_index("c")
        sid = lax.axis_index("s")
        r0 = sid * ROWS_PER_TILE
        e0 = (cid * NTILES + sid) * EPT
        pltpu.sync_copy(feat_hbm.at[pl.ds(r0, ROWS_PER_TILE)],
                        acc_sh.at[pl.ds(r0, ROWS_PER_TILE)])
        pltpu.sync_copy(edge_hbm.at[0, pl.ds(e0, EPT)], src_v)
        pltpu.sync_copy(edge_hbm.at[1, pl.ds(e0, EPT)], dst_v)
        plsc.subcore_barrier()

        def sidx(j):
            return src_v.at[pl.ds(j * CHUNK, CHUNK)]

        def didx(j):
            return dst_v.at[pl.ds(j * CHUNK, CHUNK)]

        for m in range(LOOKAHEAD):
            pltpu.async_copy(feat_hbm.at[sidx(m)], bufs[m], sg[m])

        def body(k2, carry):
            for i in range(RING):
                j = RING * k2 + i
                pltpu.make_async_copy(feat_hbm.at[sidx(j)], bufs[i],
                                      sg[i]).wait()
                pltpu.async_copy(bufs[i], acc_sh.at[didx(j)], ss[i],
                                 add=True)
                m = (i + LOOKAHEAD) % RING

                def drain_prev_scatter():
                    pltpu.make_async_copy(bufs[m], acc_sh.at[didx(0)],
                                          ss[m]).wait()

                if i >= 2:
                    drain_prev_scatter()
                else:
                    pl.when(k2 > 0)(drain_prev_scatter)
                jn = jnp.minimum(j + LOOKAHEAD, NCHUNK - 1)
                pltpu.async_copy(feat_hbm.at[sidx(jn)], bufs[m], sg[m])
            return carry

        lax.fori_loop(0, NCHUNK // RING, body, 0)
        for m in range(LOOKAHEAD):
            pltpu.make_async_copy(feat_hbm.at[sidx(NCHUNK - 1)], bufs[m],
                                  sg[m]).wait()
        for m in (RING - 2, RING - 1):
            pltpu.make_async_copy(bufs[m], acc_sh.at[didx(0)],
                                  ss[m]).wait()
        plsc.subcore_barrier()
        pltpu.sync_copy(acc_sh.at[pl.ds(r0, ROWS_PER_TILE)],
                        out_hbm.at[cid, pl.ds(r0, ROWS_PER_TILE)])

    return k(feature, edge_index)


BLK = 1000
NBLK = N // BLK


def _tc_mlp_bn(partials, feature, eps, W1, b1, W2, gamma, beta):
    def body(eps_ref, p0_ref, p1_ref, x_ref, w1_ref, b1_ref, w2_ref,
             g_ref, bt_ref, o_ref, h2_scr, s_scr, q_scr):
        ph = pl.program_id(0)
        i = pl.program_id(1)

        @pl.when(ph == 0)
        def _():
            y = (p0_ref[0] + p1_ref[0]
                 + (eps_ref[0] - 1.0) * x_ref[...])
            h = jnp.dot(y, w1_ref[...], preferred_element_type=jnp.float32)
            h = jnp.maximum(h + b1_ref[...], 0.0)
            h = jnp.dot(h, w2_ref[...], preferred_element_type=jnp.float32)
            h2_scr[pl.ds(i * BLK, BLK), :] = h

            @pl.when(i == 0)
            def _():
                s_scr[...] = jnp.zeros_like(s_scr)
                q_scr[...] = jnp.zeros_like(q_scr)

            s_scr[...] += jnp.sum(h, axis=0, keepdims=True)
            q_scr[...] += jnp.sum(h * h, axis=0, keepdims=True)

        @pl.when(ph == 1)
        def _():
            mean = s_scr[...] * (1.0 / N)
            var = q_scr[...] * (1.0 / N) - mean * mean
            scale = lax.rsqrt(var + 1e-5) * g_ref[...]
            h = h2_scr[pl.ds(i * BLK, BLK), :]
            o_ref[...] = jnp.maximum((h - mean) * scale + bt_ref[...], 0.0)

    in_p0 = pl.BlockSpec((1, BLK, D),
                         lambda ph, i: (0, ph * (NBLK - 1) + (1 - ph) * i, 0))
    in_p1 = pl.BlockSpec((1, BLK, D),
                         lambda ph, i: (1, ph * (NBLK - 1) + (1 - ph) * i, 0))
    in_row = pl.BlockSpec((BLK, D), lambda ph, i: (ph * (NBLK - 1)
                                                   + (1 - ph) * i, 0))
    out_row = pl.BlockSpec((BLK, D), lambda ph, i: (ph * i, 0))
    full_spec = pl.BlockSpec((D, D), lambda ph, i: (0, 0))
    vec_spec = pl.BlockSpec((1, D), lambda ph, i: (0, 0))
    return pl.pallas_call(
        body,
        grid=(2, NBLK),
        out_shape=jax.ShapeDtypeStruct((N, D), jnp.float32),
        in_specs=[pl.BlockSpec(memory_space=pltpu.SMEM),
                  in_p0, in_p1, in_row, full_spec, vec_spec, full_spec,
                  vec_spec, vec_spec],
        out_specs=out_row,
        scratch_shapes=[pltpu.VMEM((N, D), jnp.float32),
                        pltpu.VMEM((1, D), jnp.float32),
                        pltpu.VMEM((1, D), jnp.float32)],
    )(eps, partials, partials, feature, W1, b1.reshape(1, D), W2,
      gamma.reshape(1, D), beta.reshape(1, D))


def kernel(feature, edge_index, eps, W1, b1, W2, b2, gamma, beta):
    del b2
    partials = _sc_segment_sum(feature, edge_index)
    return _tc_mlp_bn(partials, feature, eps, W1, b1, W2, gamma, beta)


# speedup vs baseline: 1.0465x; 1.0465x over previous
import functools

import jax
import jax.numpy as jnp
from jax import lax
from jax.experimental import pallas as pl
from jax.experimental.pallas import tpu as pltpu
from jax.experimental.pallas import tpu_sc as plsc

N = 10000
E = 320000
D = 128
NTILES = 16
NWORKERS = 32
CHUNK = 40
EPT = E // NWORKERS
NCHUNK = EPT // CHUNK
ROWS_PER_TILE = N // NTILES
RING = 5
LOOKAHEAD = 3


def _sc_segment_sum(feature, edge_index):
    mesh = plsc.VectorSubcoreMesh(core_axis_name="c", subcore_axis_name="s")

    @functools.partial(
        pl.kernel,
        mesh=mesh,
        compiler_params=pltpu.CompilerParams(use_tc_tiling_on_sc=False),
        out_type=jax.ShapeDtypeStruct((2, N, D), jnp.float32),
        scratch_types=[
            pltpu.VMEM_SHARED((N, D), jnp.float32),
            pltpu.VMEM((EPT,), jnp.int32),
            pltpu.VMEM((EPT,), jnp.int32),
            [pltpu.VMEM((CHUNK, D), jnp.float32)] * RING,
            [pltpu.SemaphoreType.DMA] * RING,
            [pltpu.SemaphoreType.DMA] * RING,
        ],
    )
    def k(feat_hbm, edge_hbm, out_hbm, acc_sh, src_v, dst_v, bufs, sg, ss):
        cid = lax.axis_index("c")
        sid = lax.axis_index("s")
        r0 = sid * ROWS_PER_TILE
        e0 = (cid * NTILES + sid) * EPT
        st0 = pltpu.async_copy(feat_hbm.at[pl.ds(r0, ROWS_PER_TILE)],
                               acc_sh.at[pl.ds(r0, ROWS_PER_TILE)], sg[0])
        st1 = pltpu.async_copy(edge_hbm.at[0, pl.ds(e0, EPT)], src_v, sg[1])
        st2 = pltpu.async_copy(edge_hbm.at[1, pl.ds(e0, EPT)], dst_v, sg[2])
        st0.wait()
        st1.wait()
        st2.wait()
        plsc.subcore_barrier()

        def sidx(j):
            return src_v.at[pl.ds(j * CHUNK, CHUNK)]

        def didx(j):
            return dst_v.at[pl.ds(j * CHUNK, CHUNK)]

        for m in range(LOOKAHEAD):
            pltpu.async_copy(feat_hbm.at[sidx(m)], bufs[m], sg[m])

        def body(k2, carry):
            for i in range(RING):
                j = RING * k2 + i
                pltpu.make_async_copy(feat_hbm.at[sidx(j)], bufs[i],
                                      sg[i]).wait()
                pltpu.async_copy(bufs[i], acc_sh.at[didx(j)], ss[i],
                                 add=True)
                m = (i + LOOKAHEAD) % RING

                def drain_prev_scatter():
                    pltpu.make_async_copy(bufs[m], acc_sh.at[didx(0)],
                                          ss[m]).wait()

                if i >= 2:
                    drain_prev_scatter()
                else:
                    pl.when(k2 > 0)(drain_prev_scatter)
                jn = jnp.minimum(j + LOOKAHEAD, NCHUNK - 1)
                pltpu.async_copy(feat_hbm.at[sidx(jn)], bufs[m], sg[m])
            return carry

        lax.fori_loop(0, NCHUNK // RING, body, 0)
        for m in range(LOOKAHEAD):
            pltpu.make_async_copy(feat_hbm.at[sidx(NCHUNK - 1)], bufs[m],
                                  sg[m]).wait()
        for m in (RING - 2, RING - 1):
            pltpu.make_async_copy(bufs[m], acc_sh.at[didx(0)],
                                  ss[m]).wait()
        plsc.subcore_barrier()
        pltpu.sync_copy(acc_sh.at[pl.ds(r0, ROWS_PER_TILE)],
                        out_hbm.at[cid, pl.ds(r0, ROWS_PER_TILE)])

    return k(feature, edge_index)


def _tc_mlp_bn(partials, feature, eps, W1, b1, W2, gamma, beta):
    def body(eps_ref, p_ref, x_ref, w1_ref, b1_ref, w2_ref,
             g_ref, bt_ref, o_ref):
        y = p_ref[0] + p_ref[1] + (eps_ref[0] - 1.0) * x_ref[...]
        h = jnp.dot(y, w1_ref[...], preferred_element_type=jnp.float32)
        h = jnp.maximum(h + b1_ref[...], 0.0)
        h = jnp.dot(h, w2_ref[...], preferred_element_type=jnp.float32)
        mean = jnp.mean(h, axis=0, keepdims=True)
        d = h - mean
        var = jnp.mean(d * d, axis=0, keepdims=True)
        h = d * lax.rsqrt(var + 1e-5) * g_ref[...] + bt_ref[...]
        o_ref[...] = jnp.maximum(h, 0.0)

    vspec = pl.BlockSpec(memory_space=pltpu.VMEM)
    return pl.pallas_call(
        body,
        out_shape=jax.ShapeDtypeStruct((N, D), jnp.float32),
        in_specs=[pl.BlockSpec(memory_space=pltpu.SMEM)] + [vspec] * 7,
        out_specs=vspec,
    )(eps, partials, feature, W1, b1.reshape(1, D), W2,
      gamma.reshape(1, D), beta.reshape(1, D))


def kernel(feature, edge_index, eps, W1, b1, W2, b2, gamma, beta):
    del b2
    partials = _sc_segment_sum(feature, edge_index)
    return _tc_mlp_bn(partials, feature, eps, W1, b1, W2, gamma, beta)


# speedup vs baseline: 1.1253x; 1.0753x over previous
import functools

import jax
import jax.numpy as jnp
from jax import lax
from jax.experimental import pallas as pl
from jax.experimental.pallas import tpu as pltpu
from jax.experimental.pallas import tpu_sc as plsc

N = 10000
E = 320000
D = 128
NTILES = 16
NWORKERS = 32
CHUNK = 40
EPT = E // NWORKERS
NCHUNK = EPT // CHUNK
ROWS_PER_TILE = N // NTILES
RING = 5
LOOKAHEAD = 4


def _sc_segment_sum(feature, edge_index):
    mesh = plsc.VectorSubcoreMesh(core_axis_name="c", subcore_axis_name="s")

    @functools.partial(
        pl.kernel,
        mesh=mesh,
        compiler_params=pltpu.CompilerParams(use_tc_tiling_on_sc=False),
        out_type=jax.ShapeDtypeStruct((2, N, D), jnp.float32),
        scratch_types=[
            pltpu.VMEM_SHARED((N, D), jnp.float32),
            pltpu.VMEM((EPT,), jnp.int32),
            pltpu.VMEM((EPT,), jnp.int32),
            [pltpu.VMEM((CHUNK, D), jnp.float32)] * RING,
            [pltpu.SemaphoreType.DMA] * RING,
            [pltpu.SemaphoreType.DMA] * RING,
        ],
    )
    def k(feat_hbm, edge_hbm, out_hbm, acc_sh, src_v, dst_v, bufs, sg, ss):
        cid = lax.axis_index("c")
        sid = lax.axis_index("s")
        r0 = sid * ROWS_PER_TILE
        e0 = (cid * NTILES + sid) * EPT
        st0 = pltpu.async_copy(feat_hbm.at[pl.ds(r0, ROWS_PER_TILE)],
                               acc_sh.at[pl.ds(r0, ROWS_PER_TILE)], sg[0])
        st1 = pltpu.async_copy(edge_hbm.at[0, pl.ds(e0, EPT)], src_v, sg[1])
        st2 = pltpu.async_copy(edge_hbm.at[1, pl.ds(e0, EPT)], dst_v, sg[2])
        st0.wait()
        st1.wait()
        st2.wait()
        plsc.subcore_barrier()

        def sidx(j):
            return src_v.at[pl.ds(j * CHUNK, CHUNK)]

        def didx(j):
            return dst_v.at[pl.ds(j * CHUNK, CHUNK)]

        for m in range(LOOKAHEAD):
            pltpu.async_copy(feat_hbm.at[sidx(m)], bufs[m], sg[m])

        def body(k2, carry):
            for i in range(RING):
                j = RING * k2 + i
                pltpu.make_async_copy(feat_hbm.at[sidx(j)], bufs[i],
                                      sg[i]).wait()
                pltpu.async_copy(bufs[i], acc_sh.at[didx(j)], ss[i],
                                 add=True)
                m = (i + LOOKAHEAD) % RING

                def drain_prev_scatter():
                    pltpu.make_async_copy(bufs[m], acc_sh.at[didx(0)],
                                          ss[m]).wait()

                if i >= RING - LOOKAHEAD:
                    drain_prev_scatter()
                else:
                    pl.when(k2 > 0)(drain_prev_scatter)
                jn = jnp.minimum(j + LOOKAHEAD, NCHUNK - 1)
                pltpu.async_copy(feat_hbm.at[sidx(jn)], bufs[m], sg[m])
            return carry

        lax.fori_loop(0, NCHUNK // RING, body, 0)
        for m in range(LOOKAHEAD):
            pltpu.make_async_copy(feat_hbm.at[sidx(NCHUNK - 1)], bufs[m],
                                  sg[m]).wait()
        for m in range(LOOKAHEAD, RING):
            pltpu.make_async_copy(bufs[m], acc_sh.at[didx(0)],
                                  ss[m]).wait()
        plsc.subcore_barrier()
        pltpu.sync_copy(acc_sh.at[pl.ds(r0, ROWS_PER_TILE)],
                        out_hbm.at[cid, pl.ds(r0, ROWS_PER_TILE)])

    return k(feature, edge_index)


def _tc_mlp_bn(partials, feature, eps, W1, b1, W2, gamma, beta):
    def body(eps_ref, p_ref, x_ref, w1_ref, b1_ref, w2_ref,
             g_ref, bt_ref, o_ref):
        y = p_ref[0] + p_ref[1] + (eps_ref[0] - 1.0) * x_ref[...]
        h = jnp.dot(y, w1_ref[...], preferred_element_type=jnp.float32)
        h = jnp.maximum(h + b1_ref[...], 0.0)
        h = jnp.dot(h, w2_ref[...], preferred_element_type=jnp.float32)
        mean = jnp.mean(h, axis=0, keepdims=True)
        d = h - mean
        var = jnp.mean(d * d, axis=0, keepdims=True)
        h = d * lax.rsqrt(var + 1e-5) * g_ref[...] + bt_ref[...]
        o_ref[...] = jnp.maximum(h, 0.0)

    vspec = pl.BlockSpec(memory_space=pltpu.VMEM)
    return pl.pallas_call(
        body,
        out_shape=jax.ShapeDtypeStruct((N, D), jnp.float32),
        in_specs=[pl.BlockSpec(memory_space=pltpu.SMEM)] + [vspec] * 7,
        out_specs=vspec,
    )(eps, partials, feature, W1, b1.reshape(1, D), W2,
      gamma.reshape(1, D), beta.reshape(1, D))


def kernel(feature, edge_index, eps, W1, b1, W2, b2, gamma, beta):
    del b2
    partials = _sc_segment_sum(feature, edge_index)
    return _tc_mlp_bn(partials, feature, eps, W1, b1, W2, gamma, beta)
